# baseline (speedup 1.0000x reference)
import functools

import jax
import jax.numpy as jnp
from jax import lax
from jax.experimental import pallas as pl
from jax.experimental.pallas import tpu as pltpu
from jax.experimental.pallas import tpu_sc as plsc

N_NODES = 10000
D_FEAT = 128
N_EDGES = 320000

NC = 2
NS = 16
NW = NC * NS

CHUNK = 128
CPW = 80
E_PAD = NW * CPW * CHUNK
EPW = E_PAD // NW

NB = 3
QROWS = 4096
NV = NB * QROWS
NA = 10240
RPS = NA // NS

_mesh = plsc.VectorSubcoreMesh(core_axis_name="c", subcore_axis_name="s")


@functools.partial(
    pl.kernel,
    out_type=jax.ShapeDtypeStruct((NC, NA), jnp.float32),
    mesh=_mesh,
    scratch_types=[
        pltpu.VMEM_SHARED((NA,), jnp.float32),
        pltpu.VMEM((CPW, CHUNK), jnp.int32),
        pltpu.VMEM((CPW, CHUNK), jnp.float32),
        pltpu.VMEM((RPS,), jnp.float32),
    ],
)
def _deg_kernel(dst_hbm, deg_out, deg_sh, didx_v, ones_v, zbuf_v):
    c = lax.axis_index("c")
    s = lax.axis_index("s")
    wid = s * NC + c

    zeros16 = jnp.zeros((16,), jnp.float32)
    ones16 = jnp.full((16,), 1.0, jnp.float32)

    def _z(i, _):
        zbuf_v[pl.ds(i * 16, 16)] = zeros16
        return 0

    lax.fori_loop(0, RPS // 16, _z, 0)

    def _o(i, _):
        for j in range(CHUNK // 16):
            ones_v[i, pl.ds(j * 16, 16)] = ones16
        return 0

    lax.fori_loop(0, CPW, _o, 0)
    pltpu.sync_copy(zbuf_v, deg_sh.at[pl.ds(s * RPS, RPS)])
    pltpu.sync_copy(dst_hbm.at[wid], didx_v)
    plsc.subcore_barrier()

    def _body(t, _):
        pltpu.sync_copy(ones_v.at[t], deg_sh.at[didx_v.at[t]], add=True)
        return 0

    lax.fori_loop(0, CPW, _body, 0)
    plsc.subcore_barrier()
    pltpu.sync_copy(deg_sh.at[pl.ds(s * RPS, RPS)],
                    deg_out.at[c, pl.ds(s * RPS, RPS)])


C3 = 128
C3U = 4
CPT3 = EPW // C3


@functools.partial(
    pl.kernel,
    out_type=jax.ShapeDtypeStruct((NC, NA, D_FEAT), jnp.float32),
    mesh=_mesh,
    scratch_types=[
        pltpu.VMEM_SHARED((NA, D_FEAT), jnp.float32),
        [pltpu.VMEM((C3,), jnp.int32)] * 4,
        [pltpu.VMEM((C3,), jnp.int32)] * 2,
        [pltpu.VMEM((C3, D_FEAT), jnp.float32)] * 2,
        pltpu.SemaphoreType.DMA,
        pltpu.SemaphoreType.DMA,
        pltpu.SemaphoreType.DMA,
    ],
)
def _agg_kernel(gx_hbm, src_hbm, dst_hbm, acc_out, acc_sh, sbufs, dbufs,
                rfs, sem_g, sem_s, sem_d):
    c = lax.axis_index("c")
    s = lax.axis_index("s")
    wid = s * NC + c

    zeros16 = jnp.zeros((16,), jnp.float32)

    def _z(i, _):
        for j in range(D_FEAT // 16):
            rfs[0][i, pl.ds(j * 16, 16)] = zeros16
        return 0

    lax.fori_loop(0, C3, _z, 0)
    for k in range(RPS // C3):
        pltpu.sync_copy(rfs[0], acc_sh.at[pl.ds(s * RPS + k * C3, C3)])
    plsc.subcore_barrier()

    n = CPT3
    pltpu.sync_copy(dst_hbm.at[wid, 0], dbufs[0])
    pltpu.async_copy(src_hbm.at[wid, 0], sbufs[0], sem_s)
    pltpu.async_copy(src_hbm.at[wid, 1], sbufs[1], sem_s)
    pltpu.make_async_copy(src_hbm.at[wid, 0], sbufs[0], sem_s).wait()
    pltpu.async_copy(gx_hbm.at[sbufs[0]], rfs[0], sem_g)

    def _body(k, _):
        for b in range(C3U):
            t = C3U * k + b
            sb1 = sbufs[(b + 1) % 4]
            sb2 = sbufs[(b + 2) % 4]
            db0 = dbufs[b % 2]
            db1 = dbufs[(b + 1) % 2]
            rf0 = rfs[b % 2]
            rf1 = rfs[(b + 1) % 2]

            @pl.when(t + 2 < n)
            def _():
                pltpu.async_copy(src_hbm.at[wid, t + 2], sb2, sem_s)

            @pl.when(t + 1 < n)
            def _():
                pltpu.make_async_copy(src_hbm.at[wid, t + 1], sb1,
                                      sem_s).wait()
                pltpu.async_copy(gx_hbm.at[sb1], rf1, sem_g)
                pltpu.async_copy(dst_hbm.at[wid, t + 1], db1, sem_d)

            pltpu.make_async_copy(gx_hbm.at[sbufs[b % 4]], rf0,
                                  sem_g).wait()

            @pl.when(t > 0)
            def _():
                pltpu.make_async_copy(dst_hbm.at[wid, t], db0, sem_d).wait()

            pltpu.sync_copy(rf0, acc_sh.at[db0], add=True)
        return 0

    lax.fori_loop(0, n // C3U, _body, 0)

    plsc.subcore_barrier()
    pltpu.sync_copy(acc_sh.at[pl.ds(s * RPS, RPS)],
                    acc_out.at[c, pl.ds(s * RPS, RPS)])


def _scale_body(deg_ref, x_ref, gx_ref):
    dtot = 1.0 + deg_ref[:, 0:1] + deg_ref[:, 1:2]
    dinv = lax.rsqrt(jnp.maximum(dtot, 1e-12))
    gx_ref[...] = dinv * x_ref[...]


def _scale_kernel(degT, x_pad):
    blk = 2048
    grid = (NV // blk,)
    return pl.pallas_call(
        _scale_body,
        grid=grid,
        in_specs=[
            pl.BlockSpec((blk, 2), lambda i: (i, 0)),
            pl.BlockSpec((blk, D_FEAT), lambda i: (i, 0)),
        ],
        out_specs=pl.BlockSpec((blk, D_FEAT), lambda i: (i, 0)),
        out_shape=jax.ShapeDtypeStruct((NV, D_FEAT), jnp.float32),
    )(degT, x_pad)


def _head_body(deg_ref, acc_ref, gx_ref, x_ref, wg_ref, bg_ref, w1_ref,
               b1_ref, w2_ref, b2_ref, w3_ref, b3_ref, y_ref):
    dtot = 1.0 + deg_ref[:, 0:1] + deg_ref[:, 1:2]
    dinv = lax.rsqrt(jnp.maximum(dtot, 1e-12))
    sagg = dinv * (acc_ref[0] + acc_ref[1] + gx_ref[...])
    z = jnp.dot(sagg, wg_ref[...], preferred_element_type=jnp.float32)
    z = jnp.maximum(z + bg_ref[...], 0.0) + x_ref[...]
    y1 = jnp.dot(z, w1_ref[...], preferred_element_type=jnp.float32)
    y1 = jnp.maximum(y1 + b1_ref[...], 0.0)
    y2 = jnp.dot(y1, w2_ref[...], preferred_element_type=jnp.float32)
    y2 = jnp.maximum(y2 + b2_ref[...], 0.0)
    y_ref[...] = jnp.dot(y2, w3_ref[...],
                         preferred_element_type=jnp.float32) + b3_ref[...]


def _head_kernel(degT, acc, gx, x_pad, W_gcn, b_gcn, W1, b1, W2, b2, W3, b3):
    blk = 2048
    grid = (NA // blk,)
    full = lambda shape: pl.BlockSpec(shape, lambda i: tuple(0 for _ in shape))
    return pl.pallas_call(
        _head_body,
        grid=grid,
        in_specs=[
            pl.BlockSpec((blk, 2), lambda i: (i, 0)),
            pl.BlockSpec((NC, blk, D_FEAT), lambda i: (0, i, 0)),
            pl.BlockSpec((blk, D_FEAT), lambda i: (i, 0)),
            pl.BlockSpec((blk, D_FEAT), lambda i: (i, 0)),
            full((D_FEAT, D_FEAT)),
            full((1, D_FEAT)),
            full((D_FEAT, 32)),
            full((1, 32)),
            full((32, 32)),
            full((1, 32)),
            full((32, 4)),
            full((1, 4)),
        ],
        out_specs=pl.BlockSpec((blk, 4), lambda i: (i, 0)),
        out_shape=jax.ShapeDtypeStruct((NA, 4), jnp.float32),
    )(degT, acc, gx, x_pad, W_gcn, b_gcn, W1, b1, W2, b2, W3, b3)


def kernel(x, edge_index, W_gcn, b_gcn, W1, b1, W2, b2, W3, b3):
    src = edge_index[0].astype(jnp.int32)
    dst = edge_index[1].astype(jnp.int32)
    pad = E_PAD - N_EDGES
    srcp = jnp.concatenate([src, jnp.full((pad,), N_NODES, jnp.int32)])
    dstp = jnp.concatenate([dst, jnp.full((pad,), N_NODES, jnp.int32)])
    x_pad = jnp.pad(x, ((0, NV - N_NODES), (0, 0)))

    degp = _deg_kernel(dstp.reshape(NW, CPW, CHUNK))
    degT = jnp.pad(degp.T, ((0, NV - NA), (0, 0)))
    gx = _scale_kernel(degT, x_pad)
    acc = _agg_kernel(gx, srcp.reshape(NW, EPW // C3, C3),
                      dstp.reshape(NW, EPW // C3, C3))
    yp = _head_kernel(degT[:NA], acc, gx[:NA], x_pad[:NA], W_gcn,
                      b_gcn.reshape(1, -1), W1, b1.reshape(1, -1),
                      W2, b2.reshape(1, -1), W3, b3.reshape(1, -1))
    return yp[:N_NODES]


# speedup vs baseline: 1.1239x; 1.1239x over previous
import functools

import jax
import jax.numpy as jnp
from jax import lax
from jax.experimental import pallas as pl
from jax.experimental.pallas import tpu as pltpu
from jax.experimental.pallas import tpu_sc as plsc

N_NODES = 10000
D_FEAT = 128
N_EDGES = 320000

NC = 2
NS = 16
NW = NC * NS

CHUNK = 128
CPW = 80
E_PAD = NW * CPW * CHUNK
EPW = E_PAD // NW

NB = 3
QROWS = 4096
NV = NB * QROWS
NA = 10240
RPS = NA // NS

_mesh = plsc.VectorSubcoreMesh(core_axis_name="c", subcore_axis_name="s")


@functools.partial(
    pl.kernel,
    out_type=jax.ShapeDtypeStruct((NC, NA), jnp.float32),
    mesh=_mesh,
    scratch_types=[
        pltpu.VMEM_SHARED((NA,), jnp.float32),
        pltpu.VMEM((CPW, CHUNK), jnp.int32),
        pltpu.VMEM((CPW, CHUNK), jnp.float32),
        pltpu.VMEM((RPS,), jnp.float32),
    ],
)
def _deg_kernel(dst_hbm, deg_out, deg_sh, didx_v, ones_v, zbuf_v):
    c = lax.axis_index("c")
    s = lax.axis_index("s")
    wid = s * NC + c

    zeros16 = jnp.zeros((16,), jnp.float32)
    ones16 = jnp.full((16,), 1.0, jnp.float32)

    def _z(i, _):
        zbuf_v[pl.ds(i * 16, 16)] = zeros16
        return 0

    lax.fori_loop(0, RPS // 16, _z, 0)

    def _o(i, _):
        for j in range(CHUNK // 16):
            ones_v[i, pl.ds(j * 16, 16)] = ones16
        return 0

    lax.fori_loop(0, CPW, _o, 0)
    pltpu.sync_copy(zbuf_v, deg_sh.at[pl.ds(s * RPS, RPS)])
    pltpu.sync_copy(dst_hbm.at[wid], didx_v)
    plsc.subcore_barrier()

    def _body(t, _):
        pltpu.sync_copy(ones_v.at[t], deg_sh.at[didx_v.at[t]], add=True)
        return 0

    lax.fori_loop(0, CPW, _body, 0)
    plsc.subcore_barrier()
    pltpu.sync_copy(deg_sh.at[pl.ds(s * RPS, RPS)],
                    deg_out.at[c, pl.ds(s * RPS, RPS)])


C3 = 32
C3U = 4
CPT3 = EPW // C3


@functools.partial(
    pl.kernel,
    out_type=jax.ShapeDtypeStruct((NC, NA, D_FEAT), jnp.float32),
    mesh=_mesh,
    scratch_types=[
        pltpu.VMEM_SHARED((NA, D_FEAT), jnp.float32),
        [pltpu.VMEM((C3,), jnp.int32)] * 4,
        [pltpu.VMEM((C3,), jnp.int32)] * 2,
        [pltpu.VMEM((C3, D_FEAT), jnp.float32)] * 2,
        pltpu.SemaphoreType.DMA,
        pltpu.SemaphoreType.DMA,
        pltpu.SemaphoreType.DMA,
    ],
)
def _agg_kernel(gx_hbm, src_hbm, dst_hbm, acc_out, acc_sh, sbufs, dbufs,
                rfs, sem_g, sem_s, sem_d):
    c = lax.axis_index("c")
    s = lax.axis_index("s")
    wid = s * NC + c

    zeros16 = jnp.zeros((16,), jnp.float32)

    def _z(i, _):
        for j in range(D_FEAT // 16):
            rfs[0][i, pl.ds(j * 16, 16)] = zeros16
        return 0

    lax.fori_loop(0, C3, _z, 0)
    for k in range(RPS // C3):
        pltpu.sync_copy(rfs[0], acc_sh.at[pl.ds(s * RPS + k * C3, C3)])
    plsc.subcore_barrier()

    n = CPT3
    pltpu.sync_copy(dst_hbm.at[wid, 0], dbufs[0])
    pltpu.async_copy(src_hbm.at[wid, 0], sbufs[0], sem_s)
    pltpu.async_copy(src_hbm.at[wid, 1], sbufs[1], sem_s)
    pltpu.make_async_copy(src_hbm.at[wid, 0], sbufs[0], sem_s).wait()
    pltpu.async_copy(gx_hbm.at[sbufs[0]], rfs[0], sem_g)

    def _body(k, _):
        for b in range(C3U):
            t = C3U * k + b
            sb1 = sbufs[(b + 1) % 4]
            sb2 = sbufs[(b + 2) % 4]
            db0 = dbufs[b % 2]
            db1 = dbufs[(b + 1) % 2]
            rf0 = rfs[b % 2]
            rf1 = rfs[(b + 1) % 2]

            @pl.when(t + 2 < n)
            def _():
                pltpu.async_copy(src_hbm.at[wid, t + 2], sb2, sem_s)

            @pl.when(t + 1 < n)
            def _():
                pltpu.make_async_copy(src_hbm.at[wid, t + 1], sb1,
                                      sem_s).wait()
                pltpu.async_copy(gx_hbm.at[sb1], rf1, sem_g)
                pltpu.async_copy(dst_hbm.at[wid, t + 1], db1, sem_d)

            pltpu.make_async_copy(gx_hbm.at[sbufs[b % 4]], rf0,
                                  sem_g).wait()

            @pl.when(t > 0)
            def _():
                pltpu.make_async_copy(dst_hbm.at[wid, t], db0, sem_d).wait()

            pltpu.sync_copy(rf0, acc_sh.at[db0], add=True)
        return 0

    lax.fori_loop(0, n // C3U, _body, 0)

    plsc.subcore_barrier()
    pltpu.sync_copy(acc_sh.at[pl.ds(s * RPS, RPS)],
                    acc_out.at[c, pl.ds(s * RPS, RPS)])


def _scale_body(deg_ref, x_ref, gx_ref):
    dtot = 1.0 + deg_ref[:, 0:1] + deg_ref[:, 1:2]
    dinv = lax.rsqrt(jnp.maximum(dtot, 1e-12))
    gx_ref[...] = dinv * x_ref[...]


def _scale_kernel(degT, x_pad):
    blk = 2048
    grid = (NV // blk,)
    return pl.pallas_call(
        _scale_body,
        grid=grid,
        in_specs=[
            pl.BlockSpec((blk, 2), lambda i: (i, 0)),
            pl.BlockSpec((blk, D_FEAT), lambda i: (i, 0)),
        ],
        out_specs=pl.BlockSpec((blk, D_FEAT), lambda i: (i, 0)),
        out_shape=jax.ShapeDtypeStruct((NV, D_FEAT), jnp.float32),
    )(degT, x_pad)


def _head_body(deg_ref, acc_ref, gx_ref, x_ref, wg_ref, bg_ref, w1_ref,
               b1_ref, w2_ref, b2_ref, w3_ref, b3_ref, y_ref):
    dtot = 1.0 + deg_ref[:, 0:1] + deg_ref[:, 1:2]
    dinv = lax.rsqrt(jnp.maximum(dtot, 1e-12))
    sagg = dinv * (acc_ref[0] + acc_ref[1] + gx_ref[...])
    z = jnp.dot(sagg, wg_ref[...], preferred_element_type=jnp.float32)
    z = jnp.maximum(z + bg_ref[...], 0.0) + x_ref[...]
    y1 = jnp.dot(z, w1_ref[...], preferred_element_type=jnp.float32)
    y1 = jnp.maximum(y1 + b1_ref[...], 0.0)
    y2 = jnp.dot(y1, w2_ref[...], preferred_element_type=jnp.float32)
    y2 = jnp.maximum(y2 + b2_ref[...], 0.0)
    y_ref[...] = jnp.dot(y2, w3_ref[...],
                         preferred_element_type=jnp.float32) + b3_ref[...]


def _head_kernel(degT, acc, gx, x_pad, W_gcn, b_gcn, W1, b1, W2, b2, W3, b3):
    blk = 2048
    grid = (NA // blk,)
    full = lambda shape: pl.BlockSpec(shape, lambda i: tuple(0 for _ in shape))
    return pl.pallas_call(
        _head_body,
        grid=grid,
        in_specs=[
            pl.BlockSpec((blk, 2), lambda i: (i, 0)),
            pl.BlockSpec((NC, blk, D_FEAT), lambda i: (0, i, 0)),
            pl.BlockSpec((blk, D_FEAT), lambda i: (i, 0)),
            pl.BlockSpec((blk, D_FEAT), lambda i: (i, 0)),
            full((D_FEAT, D_FEAT)),
            full((1, D_FEAT)),
            full((D_FEAT, 32)),
            full((1, 32)),
            full((32, 32)),
            full((1, 32)),
            full((32, 4)),
            full((1, 4)),
        ],
        out_specs=pl.BlockSpec((blk, 4), lambda i: (i, 0)),
        out_shape=jax.ShapeDtypeStruct((NA, 4), jnp.float32),
    )(degT, acc, gx, x_pad, W_gcn, b_gcn, W1, b1, W2, b2, W3, b3)


def kernel(x, edge_index, W_gcn, b_gcn, W1, b1, W2, b2, W3, b3):
    src = edge_index[0].astype(jnp.int32)
    dst = edge_index[1].astype(jnp.int32)
    pad = E_PAD - N_EDGES
    srcp = jnp.concatenate([src, jnp.full((pad,), N_NODES, jnp.int32)])
    dstp = jnp.concatenate([dst, jnp.full((pad,), N_NODES, jnp.int32)])
    x_pad = jnp.pad(x, ((0, NV - N_NODES), (0, 0)))

    degp = _deg_kernel(dstp.reshape(NW, CPW, CHUNK))
    degT = jnp.pad(degp.T, ((0, NV - NA), (0, 0)))
    gx = _scale_kernel(degT, x_pad)
    acc = _agg_kernel(gx, srcp.reshape(NW, EPW // C3, C3),
                      dstp.reshape(NW, EPW // C3, C3))
    yp = _head_kernel(degT[:NA], acc, gx[:NA], x_pad[:NA], W_gcn,
                      b_gcn.reshape(1, -1), W1, b1.reshape(1, -1),
                      W2, b2.reshape(1, -1), W3, b3.reshape(1, -1))
    return yp[:N_NODES]


# speedup vs baseline: 1.1660x; 1.0374x over previous
import functools

import jax
import jax.numpy as jnp
from jax import lax
from jax.experimental import pallas as pl
from jax.experimental.pallas import tpu as pltpu
from jax.experimental.pallas import tpu_sc as plsc

N_NODES = 10000
D_FEAT = 128
N_EDGES = 320000

NC = 2
NS = 16
NW = NC * NS

CHUNK = 128
CPW = 80
E_PAD = NW * CPW * CHUNK
EPW = E_PAD // NW

NB = 3
QROWS = 4096
NV = NB * QROWS
NA = 10240
RPS = NA // NS

_mesh = plsc.VectorSubcoreMesh(core_axis_name="c", subcore_axis_name="s")


@functools.partial(
    pl.kernel,
    out_type=jax.ShapeDtypeStruct((NC, NA), jnp.float32),
    mesh=_mesh,
    scratch_types=[
        pltpu.VMEM_SHARED((NA,), jnp.float32),
        pltpu.VMEM((CPW, CHUNK), jnp.int32),
        pltpu.VMEM((CPW, CHUNK), jnp.float32),
        pltpu.VMEM((RPS,), jnp.float32),
    ],
)
def _deg_kernel(dst_hbm, deg_out, deg_sh, didx_v, ones_v, zbuf_v):
    c = lax.axis_index("c")
    s = lax.axis_index("s")
    wid = s * NC + c

    zeros16 = jnp.zeros((16,), jnp.float32)
    ones16 = jnp.full((16,), 1.0, jnp.float32)

    def _z(i, _):
        zbuf_v[pl.ds(i * 16, 16)] = zeros16
        return 0

    lax.fori_loop(0, RPS // 16, _z, 0)

    def _o(i, _):
        for j in range(CHUNK // 16):
            ones_v[i, pl.ds(j * 16, 16)] = ones16
        return 0

    lax.fori_loop(0, CPW, _o, 0)
    pltpu.sync_copy(zbuf_v, deg_sh.at[pl.ds(s * RPS, RPS)])
    pltpu.sync_copy(dst_hbm.at[wid], didx_v)
    plsc.subcore_barrier()

    def _body(t, _):
        pltpu.sync_copy(ones_v.at[t], deg_sh.at[didx_v.at[t]], add=True)
        return 0

    lax.fori_loop(0, CPW, _body, 0)
    plsc.subcore_barrier()
    pltpu.sync_copy(deg_sh.at[pl.ds(s * RPS, RPS)],
                    deg_out.at[c, pl.ds(s * RPS, RPS)])


C3 = 64
C3U = 4
CPT3 = EPW // C3


@functools.partial(
    pl.kernel,
    out_type=jax.ShapeDtypeStruct((NC, NA, D_FEAT), jnp.float32),
    mesh=_mesh,
    scratch_types=[
        pltpu.VMEM_SHARED((NA, D_FEAT), jnp.float32),
        [pltpu.VMEM((C3,), jnp.int32)] * 4,
        [pltpu.VMEM((C3,), jnp.int32)] * 2,
        [pltpu.VMEM((C3, D_FEAT), jnp.float32)] * 2,
        pltpu.SemaphoreType.DMA,
        pltpu.SemaphoreType.DMA,
        pltpu.SemaphoreType.DMA,
    ],
)
def _agg_kernel(gx_hbm, src_hbm, dst_hbm, acc_out, acc_sh, sbufs, dbufs,
                rfs, sem_g, sem_s, sem_d):
    c = lax.axis_index("c")
    s = lax.axis_index("s")
    wid = s * NC + c

    zeros16 = jnp.zeros((16,), jnp.float32)

    def _z(i, _):
        for j in range(D_FEAT // 16):
            rfs[0][i, pl.ds(j * 16, 16)] = zeros16
        return 0

    lax.fori_loop(0, C3, _z, 0)
    for k in range(RPS // C3):
        pltpu.sync_copy(rfs[0], acc_sh.at[pl.ds(s * RPS + k * C3, C3)])
    plsc.subcore_barrier()

    n = CPT3
    pltpu.sync_copy(dst_hbm.at[wid, 0], dbufs[0])
    pltpu.async_copy(src_hbm.at[wid, 0], sbufs[0], sem_s)
    pltpu.async_copy(src_hbm.at[wid, 1], sbufs[1], sem_s)
    pltpu.make_async_copy(src_hbm.at[wid, 0], sbufs[0], sem_s).wait()
    pltpu.async_copy(gx_hbm.at[sbufs[0]], rfs[0], sem_g)

    def _body(k, _):
        for b in range(C3U):
            t = C3U * k + b
            sb1 = sbufs[(b + 1) % 4]
            sb2 = sbufs[(b + 2) % 4]
            db0 = dbufs[b % 2]
            db1 = dbufs[(b + 1) % 2]
            rf0 = rfs[b % 2]
            rf1 = rfs[(b + 1) % 2]

            @pl.when(t + 2 < n)
            def _():
                pltpu.async_copy(src_hbm.at[wid, t + 2], sb2, sem_s)

            @pl.when(t + 1 < n)
            def _():
                pltpu.make_async_copy(src_hbm.at[wid, t + 1], sb1,
                                      sem_s).wait()
                pltpu.async_copy(gx_hbm.at[sb1], rf1, sem_g)
                pltpu.async_copy(dst_hbm.at[wid, t + 1], db1, sem_d)

            pltpu.make_async_copy(gx_hbm.at[sbufs[b % 4]], rf0,
                                  sem_g).wait()

            @pl.when(t > 0)
            def _():
                pltpu.make_async_copy(dst_hbm.at[wid, t], db0, sem_d).wait()

            pltpu.sync_copy(rf0, acc_sh.at[db0], add=True)
        return 0

    lax.fori_loop(0, n // C3U, _body, 0)

    plsc.subcore_barrier()
    pltpu.sync_copy(acc_sh.at[pl.ds(s * RPS, RPS)],
                    acc_out.at[c, pl.ds(s * RPS, RPS)])


def _scale_body(deg_ref, x_ref, gx_ref):
    dtot = 1.0 + deg_ref[:, 0:1] + deg_ref[:, 1:2]
    dinv = lax.rsqrt(jnp.maximum(dtot, 1e-12))
    gx_ref[...] = dinv * x_ref[...]


def _scale_kernel(degT, x_pad):
    blk = 2048
    grid = (NV // blk,)
    return pl.pallas_call(
        _scale_body,
        grid=grid,
        in_specs=[
            pl.BlockSpec((blk, 2), lambda i: (i, 0)),
            pl.BlockSpec((blk, D_FEAT), lambda i: (i, 0)),
        ],
        out_specs=pl.BlockSpec((blk, D_FEAT), lambda i: (i, 0)),
        out_shape=jax.ShapeDtypeStruct((NV, D_FEAT), jnp.float32),
    )(degT, x_pad)


def _head_body(deg_ref, acc_ref, gx_ref, x_ref, wg_ref, bg_ref, w1_ref,
               b1_ref, w2_ref, b2_ref, w3_ref, b3_ref, y_ref):
    dtot = 1.0 + deg_ref[:, 0:1] + deg_ref[:, 1:2]
    dinv = lax.rsqrt(jnp.maximum(dtot, 1e-12))
    sagg = dinv * (acc_ref[0] + acc_ref[1] + gx_ref[...])
    z = jnp.dot(sagg, wg_ref[...], preferred_element_type=jnp.float32)
    z = jnp.maximum(z + bg_ref[...], 0.0) + x_ref[...]
    y1 = jnp.dot(z, w1_ref[...], preferred_element_type=jnp.float32)
    y1 = jnp.maximum(y1 + b1_ref[...], 0.0)
    y2 = jnp.dot(y1, w2_ref[...], preferred_element_type=jnp.float32)
    y2 = jnp.maximum(y2 + b2_ref[...], 0.0)
    y_ref[...] = jnp.dot(y2, w3_ref[...],
                         preferred_element_type=jnp.float32) + b3_ref[...]


def _head_kernel(degT, acc, gx, x_pad, W_gcn, b_gcn, W1, b1, W2, b2, W3, b3):
    blk = 2048
    grid = (NA // blk,)
    full = lambda shape: pl.BlockSpec(shape, lambda i: tuple(0 for _ in shape))
    return pl.pallas_call(
        _head_body,
        grid=grid,
        in_specs=[
            pl.BlockSpec((blk, 2), lambda i: (i, 0)),
            pl.BlockSpec((NC, blk, D_FEAT), lambda i: (0, i, 0)),
            pl.BlockSpec((blk, D_FEAT), lambda i: (i, 0)),
            pl.BlockSpec((blk, D_FEAT), lambda i: (i, 0)),
            full((D_FEAT, D_FEAT)),
            full((1, D_FEAT)),
            full((D_FEAT, 32)),
            full((1, 32)),
            full((32, 32)),
            full((1, 32)),
            full((32, 4)),
            full((1, 4)),
        ],
        out_specs=pl.BlockSpec((blk, 4), lambda i: (i, 0)),
        out_shape=jax.ShapeDtypeStruct((NA, 4), jnp.float32),
    )(degT, acc, gx, x_pad, W_gcn, b_gcn, W1, b1, W2, b2, W3, b3)


def kernel(x, edge_index, W_gcn, b_gcn, W1, b1, W2, b2, W3, b3):
    src = edge_index[0].astype(jnp.int32)
    dst = edge_index[1].astype(jnp.int32)
    pad = E_PAD - N_EDGES
    srcp = jnp.concatenate([src, jnp.full((pad,), N_NODES, jnp.int32)])
    dstp = jnp.concatenate([dst, jnp.full((pad,), N_NODES, jnp.int32)])
    x_pad = jnp.pad(x, ((0, NV - N_NODES), (0, 0)))

    degp = _deg_kernel(dstp.reshape(NW, CPW, CHUNK))
    degT = jnp.pad(degp.T, ((0, NV - NA), (0, 0)))
    gx = _scale_kernel(degT, x_pad)
    acc = _agg_kernel(gx, srcp.reshape(NW, EPW // C3, C3),
                      dstp.reshape(NW, EPW // C3, C3))
    yp = _head_kernel(degT[:NA], acc, gx[:NA], x_pad[:NA], W_gcn,
                      b_gcn.reshape(1, -1), W1, b1.reshape(1, -1),
                      W2, b2.reshape(1, -1), W3, b3.reshape(1, -1))
    return yp[:N_NODES]
